# baseline (speedup 1.0000x reference)
import functools

import jax
import jax.numpy as jnp
from jax.experimental import pallas as pl
from jax.experimental.pallas import tpu as pltpu

_FEAT = 3 * 32 * 32
_NBUF = 2


def _body(it_ref, cur_ref, oh_ref, out_ref, oh_out_ref,
          ibufs, obufs, isems, osems, oh_sem, *, nchunks, bs):
    it = it_ref[0]
    oh_cp = pltpu.make_async_copy(oh_ref.at[it], oh_out_ref, oh_sem)
    oh_cp.start()

    def in_copy(i, slot):
        return pltpu.make_async_copy(
            cur_ref.at[it, pl.ds(i * bs, bs), :], ibufs.at[slot], isems.at[slot])

    def out_copy(i, slot):
        return pltpu.make_async_copy(
            obufs.at[slot], out_ref.at[pl.ds(i * bs, bs), :], osems.at[slot])

    in_copy(0, 0).start()
    for i in range(nchunks):
        slot = i % _NBUF
        if i + 1 < nchunks:
            in_copy(i + 1, (i + 1) % _NBUF).start()
        in_copy(i, slot).wait()
        if i >= _NBUF:
            out_copy(i - _NBUF, slot).wait()
        obufs[slot, :, :] = jnp.tanh(ibufs[slot, :, :]) * 2.0
        out_copy(i, slot).start()
    for i in range(max(nchunks - _NBUF, 0), nchunks):
        out_copy(i, i % _NBUF).wait()
    oh_cp.wait()


def kernel(curriculum, curriculum_labels_one_hot, it):
    n, b = curriculum.shape[0], curriculum.shape[1]
    img_shape = curriculum.shape[2:]
    nc = curriculum_labels_one_hot.shape[-1]
    cur = curriculum.reshape(n, b, _FEAT)
    oh = curriculum_labels_one_hot.reshape(n, 1, b * nc)
    it_arr = jnp.atleast_1d(jnp.asarray(it, jnp.int32))
    nchunks = 8
    bs = b // nchunks
    out, oh_out = pl.pallas_call(
        functools.partial(_body, nchunks=nchunks, bs=bs),
        in_specs=[
            pl.BlockSpec(memory_space=pltpu.SMEM),
            pl.BlockSpec(memory_space=pl.ANY),
            pl.BlockSpec(memory_space=pl.ANY),
        ],
        out_specs=[
            pl.BlockSpec(memory_space=pl.ANY),
            pl.BlockSpec(memory_space=pl.ANY),
        ],
        out_shape=[
            jax.ShapeDtypeStruct((b, _FEAT), jnp.float32),
            jax.ShapeDtypeStruct((1, b * nc), jnp.float32),
        ],
        scratch_shapes=[
            pltpu.VMEM((_NBUF, bs, _FEAT), jnp.float32),
            pltpu.VMEM((_NBUF, bs, _FEAT), jnp.float32),
            pltpu.SemaphoreType.DMA((_NBUF,)),
            pltpu.SemaphoreType.DMA((_NBUF,)),
            pltpu.SemaphoreType.DMA,
        ],
    )(it_arr, cur, oh)
    return out.reshape((b,) + img_shape), oh_out.reshape(b, nc)


# speedup vs baseline: 13.3740x; 13.3740x over previous
import jax
import jax.numpy as jnp
from jax.experimental import pallas as pl
from jax.experimental.pallas import tpu as pltpu

_FEAT = 3 * 32 * 32


def _body(it_ref, cur_ref, oh_ref, out_ref, oh_out_ref):
    del it_ref
    out_ref[...] = jnp.tanh(cur_ref[0]) * 2.0
    oh_out_ref[...] = oh_ref[:, 0]


def kernel(curriculum, curriculum_labels_one_hot, it):
    n, b = curriculum.shape[0], curriculum.shape[1]
    c, h, w = curriculum.shape[2:]
    nc = curriculum_labels_one_hot.shape[-1]
    cur_t = jnp.transpose(curriculum, (0, 2, 3, 4, 1)).reshape(n, _FEAT, b)
    oh_t = jnp.transpose(curriculum_labels_one_hot, (2, 0, 1)).reshape(nc, n, 1, b)
    it_arr = jnp.atleast_1d(jnp.asarray(it, jnp.int32))
    grid = 8
    bf = _FEAT // grid
    out_t, oh_out_t = pl.pallas_call(
        _body,
        grid_spec=pltpu.PrefetchScalarGridSpec(
            num_scalar_prefetch=1,
            grid=(grid,),
            in_specs=[
                pl.BlockSpec((1, bf, b), lambda i, it_ref: (it_ref[0], i, 0)),
                pl.BlockSpec((nc, 1, 1, b), lambda i, it_ref: (0, it_ref[0], 0, 0)),
            ],
            out_specs=[
                pl.BlockSpec((bf, b), lambda i, it_ref: (i, 0)),
                pl.BlockSpec((nc, 1, b), lambda i, it_ref: (0, 0, 0)),
            ],
        ),
        out_shape=[
            jax.ShapeDtypeStruct((_FEAT, b), jnp.float32),
            jax.ShapeDtypeStruct((nc, 1, b), jnp.float32),
        ],
    )(it_arr, cur_t, oh_t)
    out = jnp.transpose(out_t.reshape(c, h, w, b), (3, 0, 1, 2))
    oh_out = jnp.transpose(oh_out_t.reshape(nc, b), (1, 0))
    return out, oh_out


# speedup vs baseline: 35.1081x; 2.6251x over previous
import functools

import jax
import jax.numpy as jnp
from jax.experimental import pallas as pl
from jax.experimental.pallas import tpu as pltpu

_FEAT = 3 * 32 * 32
_NCH = 8


def _body(it_ref, cur_ref, oh_ref, out_ref, oh_out_ref,
          ibuf, obuf, isems, osems, oh_sem, *, bf):
    it = it_ref[0]
    oh_cp = pltpu.make_async_copy(oh_ref.at[:, it], oh_out_ref, oh_sem)
    oh_cp.start()
    for i in range(_NCH):
        pltpu.make_async_copy(
            cur_ref.at[it, pl.ds(i * bf, bf), :], ibuf.at[i], isems.at[i]
        ).start()
    for i in range(_NCH):
        pltpu.make_async_copy(
            cur_ref.at[it, pl.ds(i * bf, bf), :], ibuf.at[i], isems.at[i]
        ).wait()
        obuf[i] = jnp.tanh(ibuf[i]) * 2.0
        pltpu.make_async_copy(
            obuf.at[i], out_ref.at[pl.ds(i * bf, bf), :], osems.at[i]
        ).start()
    for i in range(_NCH):
        pltpu.make_async_copy(
            obuf.at[i], out_ref.at[pl.ds(i * bf, bf), :], osems.at[i]
        ).wait()
    oh_cp.wait()


def kernel(curriculum, curriculum_labels_one_hot, it):
    n, b = curriculum.shape[0], curriculum.shape[1]
    c, h, w = curriculum.shape[2:]
    nc = curriculum_labels_one_hot.shape[-1]
    cur_t = jnp.transpose(curriculum, (0, 2, 3, 4, 1)).reshape(n, _FEAT, b)
    oh_t = jnp.transpose(curriculum_labels_one_hot, (2, 0, 1))
    it_arr = jnp.atleast_1d(jnp.asarray(it, jnp.int32))
    bf = _FEAT // _NCH
    out_t, oh_out_t = pl.pallas_call(
        functools.partial(_body, bf=bf),
        in_specs=[
            pl.BlockSpec(memory_space=pltpu.SMEM),
            pl.BlockSpec(memory_space=pl.ANY),
            pl.BlockSpec(memory_space=pl.ANY),
        ],
        out_specs=[
            pl.BlockSpec(memory_space=pl.ANY),
            pl.BlockSpec(memory_space=pl.ANY),
        ],
        out_shape=[
            jax.ShapeDtypeStruct((_FEAT, b), jnp.float32),
            jax.ShapeDtypeStruct((nc, b), jnp.float32),
        ],
        scratch_shapes=[
            pltpu.VMEM((_NCH, bf, b), jnp.float32),
            pltpu.VMEM((_NCH, bf, b), jnp.float32),
            pltpu.SemaphoreType.DMA((_NCH,)),
            pltpu.SemaphoreType.DMA((_NCH,)),
            pltpu.SemaphoreType.DMA,
        ],
    )(it_arr, cur_t, oh_t)
    out = jnp.transpose(out_t.reshape(c, h, w, b), (3, 0, 1, 2))
    oh_out = jnp.transpose(oh_out_t, (1, 0))
    return out, oh_out
